# Initial kernel scaffold; baseline (speedup 1.0000x reference)
import jax
import jax.numpy as jnp
from jax.experimental import pallas as pl


def kernel(inputs, targets):
    raise NotImplementedError("write your pallas kernel here")


# speedup vs baseline: 20.3381x; 20.3381x over previous
import functools

import jax
import jax.numpy as jnp
from jax.experimental import pallas as pl
from jax.experimental.pallas import tpu as pltpu


def _select_body(x_ref, y_ref, out_ref, keys_ref, *, n_rows, n_cols, k, chunk):
    n_chunks = n_rows // chunk

    def compute_chunk(i, _):
        sl = pl.ds(i * chunk, chunk)
        d = x_ref[sl, :] - y_ref[sl, :]
        a = jnp.abs(d)
        loss = jnp.where(a < 1.0, 0.5 * d * d, a - 0.5)
        keys_ref[sl, :] = jax.lax.bitcast_convert_type(loss, jnp.int32)
        return 0
    jax.lax.fori_loop(0, n_chunks, compute_chunk, 0)

    def bit_step(i, prefix):
        trial = prefix + (jnp.int32(1) << (jnp.int32(30) - i))

        def cnt_chunk(j, acc):
            sl = pl.ds(j * chunk, chunk)
            return acc + jnp.sum((keys_ref[sl, :] >= trial).astype(jnp.int32))

        cnt = jax.lax.fori_loop(0, n_chunks, cnt_chunk, jnp.int32(0))
        return jnp.where(cnt >= k, trial, prefix)

    t = jax.lax.fori_loop(0, 31, bit_step, jnp.int32(0))

    def fin_chunk(j, carry):
        n, s = carry
        sl = pl.ds(j * chunk, chunk)
        kk = keys_ref[sl, :]
        gt = kk > t
        n = n + jnp.sum(gt.astype(jnp.int32))
        s = s + jnp.sum(jnp.where(gt, jax.lax.bitcast_convert_type(kk, jnp.float32), 0.0))
        return (n, s)

    n_gt, s_gt = jax.lax.fori_loop(0, n_chunks, fin_chunk,
                                   (jnp.int32(0), jnp.float32(0.0)))
    t_val = jax.lax.bitcast_convert_type(t, jnp.float32)
    out_ref[0, 0] = (s_gt + (k - n_gt).astype(jnp.float32) * t_val) / jnp.float32(k)


def kernel(inputs, targets):
    n_total = inputs.size
    k = int(0.6 * n_total)
    n_cols = inputs.shape[-1]
    n_rows = n_total // n_cols
    x = inputs.reshape(n_rows, n_cols)
    y = targets.reshape(n_rows, n_cols)
    chunk = 512 if n_rows % 512 == 0 else 8

    out = pl.pallas_call(
        functools.partial(_select_body, n_rows=n_rows, n_cols=n_cols, k=k,
                          chunk=chunk),
        out_shape=jax.ShapeDtypeStruct((1, 1), jnp.float32),
        in_specs=[pl.BlockSpec(memory_space=pltpu.VMEM),
                  pl.BlockSpec(memory_space=pltpu.VMEM)],
        out_specs=pl.BlockSpec(memory_space=pltpu.SMEM),
        scratch_shapes=[pltpu.VMEM((n_rows, n_cols), jnp.int32)],
    )(x, y)
    return out[0, 0]
